# Initial kernel scaffold; baseline (speedup 1.0000x reference)
import jax
import jax.numpy as jnp
from jax.experimental import pallas as pl


def kernel(state, inputs, perm_mats):
    raise NotImplementedError("write your pallas kernel here")


# speedup vs baseline: 59.6876x; 59.6876x over previous
import jax
import jax.numpy as jnp
from jax import lax
from jax.experimental import pallas as pl
from jax.experimental.pallas import tpu as pltpu
from jax.experimental.pallas import tpu_sc as plsc

_B = 128
_T = 8192
_LANES = 16
_CHUNK = _T // _LANES
_NC = 2
_NS = 16
_NW = _NC * _NS
_ROWS_PER_W = _B // _NW
_OUT_W = _T * 5

_ID_CODE = 0
for _i in range(5):
    _ID_CODE |= (5 * _i) << (5 * _i)


def _compose(prefix, new):
    acc = None
    for i in range(5):
        t = (new >> (5 * i)) & 31
        s = (prefix >> t) & 31
        term = s << (5 * i)
        acc = term if acc is None else acc | term
    return acc


def _sc_body(state_hbm, seq_hbm, ctab_hbm, out_hbm,
             seq_v, codes_v, out_v, state_v, ctab_v, lane_v):
    wid = lax.axis_index("s") * _NC + lax.axis_index("c")
    iota = lax.iota(jnp.int32, _LANES)
    idvec = jnp.full((_LANES,), _ID_CODE, dtype=jnp.int32)
    obase = iota * (_CHUNK * 5)

    pltpu.sync_copy(ctab_hbm, ctab_v)
    lane_v[pl.ds(0, _LANES)] = idvec

    for j in range(_ROWS_PER_W):
        row = wid * _ROWS_PER_W + j
        pltpu.sync_copy(seq_hbm.at[row], seq_v)
        pltpu.sync_copy(state_hbm.at[row], state_v)

        def pass1(k, carry):
            u = seq_v[pl.ds(k * _LANES, _LANES)]
            cu = plsc.load_gather(ctab_v, [u])
            carry = _compose(carry, cu)
            codes_v[pl.ds(k * _LANES, _LANES)] = carry
            return carry

        tot = lax.fori_loop(0, _CHUNK, pass1, idvec)

        x = tot
        for off in (1, 2, 4, 8):
            lane_v[pl.ds(_LANES, _LANES)] = x
            sh = plsc.load_gather(lane_v, [iota + (_LANES - off)])
            x = _compose(sh, x)
        lane_v[pl.ds(_LANES, _LANES)] = x
        lane_off = plsc.load_gather(lane_v, [iota + (_LANES - 1)])

        def pass2(k, carry):
            local = codes_v[pl.ds(k * _LANES, _LANES)]
            fin = _compose(lane_off, local)
            for i in range(5):
                d5 = (fin >> (5 * i)) & 31
                val = plsc.load_gather(state_v, [d5])
                plsc.store_scatter(out_v, [obase + (k * 5 + i)], val)
            return carry

        lax.fori_loop(0, _CHUNK, pass2, 0)
        pltpu.sync_copy(out_v, out_hbm.at[row])


def kernel(state, inputs, perm_mats):
    p = jnp.argmax(perm_mats, axis=2).astype(jnp.int32)
    shifts = 5 * jnp.arange(5, dtype=jnp.int32)
    codes = jnp.sum((p * 5) << shifts[None, :], axis=1).astype(jnp.int32)
    ctab = jnp.zeros((128,), jnp.int32).at[:120].set(codes)
    state_pad = jnp.zeros((_B, 32), jnp.float32).at[:, 0:25:5].set(state)
    seq = inputs.reshape(_B, _LANES, _CHUNK).swapaxes(1, 2).reshape(_B, _T)

    mesh = plsc.VectorSubcoreMesh(core_axis_name="c", subcore_axis_name="s")
    fn = pl.kernel(
        _sc_body,
        mesh=mesh,
        compiler_params=pltpu.CompilerParams(needs_layout_passes=False),
        out_type=jax.ShapeDtypeStruct((_B, _OUT_W), jnp.float32),
        scratch_types=[
            pltpu.VMEM((_T,), jnp.int32),
            pltpu.VMEM((_T,), jnp.int32),
            pltpu.VMEM((_OUT_W,), jnp.float32),
            pltpu.VMEM((32,), jnp.float32),
            pltpu.VMEM((128,), jnp.int32),
            pltpu.VMEM((32,), jnp.int32),
        ],
    )
    out = fn(state_pad, seq, ctab)
    return out.reshape(_B, _T, 5)
